# Initial kernel scaffold; baseline (speedup 1.0000x reference)
import jax
import jax.numpy as jnp
from jax.experimental import pallas as pl


def kernel(x, edge_index, edge_attr, W1, b1, g1, be1, W2, b2, g2, be2, W3, b3, x0, ap, bp, mu, log_var, We1, eb1, We2, eb2, We3, eb3):
    raise NotImplementedError("write your pallas kernel here")


# speedup vs baseline: 1.4040x; 1.4040x over previous
import functools
import math

import jax
import jax.numpy as jnp
from jax.experimental import pallas as pl

N = 10000
E = 320000
D_FEAT = 128
D_HID = 256
D_LAT = 16
N_CLASSES = 8
N_LAYERS = 10
EDGE_DIM = 16
EDGE_HID = 32
APPNP_K = 10
APPNP_ALPHA = 0.1
BETA_PRIOR = 0.001
LOG_SCALE = 0.5 * D_LAT * math.log(4 * math.pi)

_RN = 1000
_NBN = N // _RN
_EP = 41000
_RE = 1000
_NBE = _EP // _RE

_F32 = jnp.float32


def _softplus(v):
    return jnp.maximum(v, 0.0) + jnp.log1p(jnp.exp(-jnp.abs(v)))


def _enc1_body(x_ref, w_ref, b_ref, a_ref, s_ref):
    i = pl.program_id(0)

    @pl.when(i == 0)
    def _():
        s_ref[...] = jnp.zeros_like(s_ref)

    a = jnp.dot(x_ref[...], w_ref[...], preferred_element_type=_F32) + b_ref[...]
    a_ref[...] = a
    ssum = jnp.sum(a, axis=0, keepdims=True)
    ssq = jnp.sum(a * a, axis=0, keepdims=True)
    pad = jnp.zeros((6, a.shape[1]), _F32)
    s_ref[...] = s_ref[...] + jnp.concatenate([ssum, ssq, pad], axis=0)


def _enc2_body(a_ref, s_ref, g_ref, be_ref, w_ref, b_ref, a2_ref, s2_ref):
    i = pl.program_id(0)

    @pl.when(i == 0)
    def _():
        s2_ref[...] = jnp.zeros_like(s2_ref)

    mean = s_ref[0:1, :] / N
    var = s_ref[1:2, :] / N - mean * mean
    rstd = jax.lax.rsqrt(var + 1e-5)
    h = jnp.maximum((a_ref[...] - mean) * rstd * g_ref[...] + be_ref[...], 0.0)
    a2 = jnp.dot(h, w_ref[...], preferred_element_type=_F32) + b_ref[...]
    a2_ref[...] = a2
    ssum = jnp.sum(a2, axis=0, keepdims=True)
    ssq = jnp.sum(a2 * a2, axis=0, keepdims=True)
    pad = jnp.zeros((6, a2.shape[1]), _F32)
    s2_ref[...] = s2_ref[...] + jnp.concatenate([ssum, ssq, pad], axis=0)


def _flow_body(a_ref, s_ref, g_ref, be_ref, w_ref, b_ref, x0_ref, ap_ref,
               bp_ref, mu_ref, lv_ref, G_ref, H_ref, T_ref,
               z_ref, lq_ref, bt_ref):
    mean = s_ref[0:1, :] / N
    var = s_ref[1:2, :] / N - mean * mean
    rstd = jax.lax.rsqrt(var + 1e-5)
    h = jnp.maximum((a_ref[...] - mean) * rstd * g_ref[...] + be_ref[...], 0.0)
    z = jnp.dot(h, w_ref[...], preferred_element_type=_F32) + b_ref[...]
    z_ref[...] = z

    G = G_ref[...]
    H = H_ref[...]
    zz = jnp.dot(z, T_ref[...], preferred_element_type=_F32)
    ldsum = jnp.zeros((zz.shape[0], N_CLASSES), _F32)
    for l in range(N_LAYERS - 1, -1, -1):
        a8 = _softplus(ap_ref[l:l + 1, :])
        b8 = -a8 + _softplus(bp_ref[l:l + 1, :])
        diff = zz - x0_ref[l:l + 1, :]
        r2 = jnp.dot(diff * diff, G, preferred_element_type=_F32)
        r = jnp.maximum(jnp.sqrt(r2), 1e-8)
        hr = 1.0 / (a8 + r)
        bhr = b8 * hr
        zz = zz + jnp.dot(bhr, H, preferred_element_type=_F32) * diff
        ld = ((D_LAT - 1) * jnp.log1p(bhr)
              + jnp.log1p(bhr - b8 * hr * hr * r))
        ldsum = ldsum + ld
    lv = lv_ref[...]
    slv8 = jnp.dot(lv, G, preferred_element_type=_F32)
    dm = zz - mu_ref[...]
    q = jnp.dot(dm * dm * jnp.exp(-lv), G, preferred_element_type=_F32)
    logp = (-0.5 * D_LAT * math.log(2 * math.pi)) - 0.5 * slv8 - 0.5 * q
    log_q = logp + ldsum
    lq_ref[...] = log_q
    bt_ref[...] = jnp.exp(jnp.clip(log_q + LOG_SCALE, -30.0, 30.0))


def _edge_mlp_body(ea_ref, w1_ref, b1_ref, w2_ref, b2_ref, w3_ref, b3_ref, o_ref):
    h = jnp.maximum(jnp.dot(ea_ref[...], w1_ref[...], preferred_element_type=_F32)
                    + b1_ref[...], 0.0)
    h = jnp.maximum(jnp.dot(h, w2_ref[...], preferred_element_type=_F32)
                    + b2_ref[...], 0.0)
    lo = jnp.dot(h, w3_ref[...], preferred_element_type=_F32) + b3_ref[...]
    o_ref[...] = _softplus(lo)


def _dense_stage(x, W1, b1, g1, be1, W2, b2, g2, be2, W3, b3,
                 x0, ap, bp, mu, log_var, We1, eb1, We2, eb2, We3, eb3,
                 edge_attr):
    R = _RN
    f = _F32
    s_shape = jax.ShapeDtypeStruct((8, D_HID), f)

    a1, s1 = pl.pallas_call(
        _enc1_body,
        grid=(_NBN,),
        in_specs=[
            pl.BlockSpec((R, D_FEAT), lambda i: (i, 0)),
            pl.BlockSpec((D_FEAT, D_HID), lambda i: (0, 0)),
            pl.BlockSpec((1, D_HID), lambda i: (0, 0)),
        ],
        out_specs=[
            pl.BlockSpec((R, D_HID), lambda i: (i, 0)),
            pl.BlockSpec((8, D_HID), lambda i: (0, 0)),
        ],
        out_shape=[jax.ShapeDtypeStruct((N, D_HID), f), s_shape],
    )(x, W1, b1.reshape(1, -1))

    a2, s2 = pl.pallas_call(
        _enc2_body,
        grid=(_NBN,),
        in_specs=[
            pl.BlockSpec((R, D_HID), lambda i: (i, 0)),
            pl.BlockSpec((8, D_HID), lambda i: (0, 0)),
            pl.BlockSpec((1, D_HID), lambda i: (0, 0)),
            pl.BlockSpec((1, D_HID), lambda i: (0, 0)),
            pl.BlockSpec((D_HID, D_HID), lambda i: (0, 0)),
            pl.BlockSpec((1, D_HID), lambda i: (0, 0)),
        ],
        out_specs=[
            pl.BlockSpec((R, D_HID), lambda i: (i, 0)),
            pl.BlockSpec((8, D_HID), lambda i: (0, 0)),
        ],
        out_shape=[jax.ShapeDtypeStruct((N, D_HID), f), s_shape],
    )(a1, s1, g1.reshape(1, -1), be1.reshape(1, -1), W2, b2.reshape(1, -1))

    eye16 = jnp.eye(D_LAT, dtype=f)
    G = jnp.kron(jnp.eye(N_CLASSES, dtype=f), jnp.ones((D_LAT, 1), f))
    H = G.T
    T16 = jnp.kron(jnp.ones((1, N_CLASSES), f), eye16)
    x0p = jnp.transpose(x0, (1, 0, 2)).reshape(N_LAYERS, N_CLASSES * D_LAT)
    x0p = jnp.concatenate([x0p, jnp.zeros((16 - N_LAYERS, 128), f)], axis=0)
    app = jnp.concatenate([ap.T, jnp.zeros((16 - N_LAYERS, N_CLASSES), f)], 0)
    bpp = jnp.concatenate([bp.T, jnp.zeros((16 - N_LAYERS, N_CLASSES), f)], 0)
    mu128 = mu.reshape(1, -1)
    lv128 = log_var.reshape(1, -1)

    z, log_q, beta = pl.pallas_call(
        _flow_body,
        grid=(_NBN,),
        in_specs=[
            pl.BlockSpec((R, D_HID), lambda i: (i, 0)),
            pl.BlockSpec((8, D_HID), lambda i: (0, 0)),
            pl.BlockSpec((1, D_HID), lambda i: (0, 0)),
            pl.BlockSpec((1, D_HID), lambda i: (0, 0)),
            pl.BlockSpec((D_HID, D_LAT), lambda i: (0, 0)),
            pl.BlockSpec((1, D_LAT), lambda i: (0, 0)),
            pl.BlockSpec((16, 128), lambda i: (0, 0)),
            pl.BlockSpec((16, N_CLASSES), lambda i: (0, 0)),
            pl.BlockSpec((16, N_CLASSES), lambda i: (0, 0)),
            pl.BlockSpec((1, 128), lambda i: (0, 0)),
            pl.BlockSpec((1, 128), lambda i: (0, 0)),
            pl.BlockSpec((128, N_CLASSES), lambda i: (0, 0)),
            pl.BlockSpec((N_CLASSES, 128), lambda i: (0, 0)),
            pl.BlockSpec((D_LAT, 128), lambda i: (0, 0)),
        ],
        out_specs=[
            pl.BlockSpec((R, D_LAT), lambda i: (i, 0)),
            pl.BlockSpec((R, N_CLASSES), lambda i: (i, 0)),
            pl.BlockSpec((R, N_CLASSES), lambda i: (i, 0)),
        ],
        out_shape=[
            jax.ShapeDtypeStruct((N, D_LAT), f),
            jax.ShapeDtypeStruct((N, N_CLASSES), f),
            jax.ShapeDtypeStruct((N, N_CLASSES), f),
        ],
    )(a2, s2, g2.reshape(1, -1), be2.reshape(1, -1), W3, b3.reshape(1, -1),
      x0p, app, bpp, mu128, lv128, G, H, T16)

    eap = edge_attr.reshape(E // 8, 8 * EDGE_DIM)
    eap = jnp.concatenate([eap, jnp.zeros((_EP - E // 8, 8 * EDGE_DIM), f)], 0)
    eye8 = jnp.eye(8, dtype=f)
    W1bd = jnp.kron(eye8, We1)
    W2bd = jnp.kron(eye8, We2)
    W3bd = jnp.kron(eye8, We3)
    b1r = jnp.tile(eb1, 8).reshape(1, -1)
    b2r = jnp.tile(eb2, 8).reshape(1, -1)
    b3r = jnp.tile(eb3, 8).reshape(1, -1)

    ews = pl.pallas_call(
        _edge_mlp_body,
        grid=(_NBE,),
        in_specs=[
            pl.BlockSpec((_RE, 128), lambda i: (i, 0)),
            pl.BlockSpec((128, 256), lambda i: (0, 0)),
            pl.BlockSpec((1, 256), lambda i: (0, 0)),
            pl.BlockSpec((256, 256), lambda i: (0, 0)),
            pl.BlockSpec((1, 256), lambda i: (0, 0)),
            pl.BlockSpec((256, 8), lambda i: (0, 0)),
            pl.BlockSpec((1, 8), lambda i: (0, 0)),
        ],
        out_specs=[pl.BlockSpec((_RE, 8), lambda i: (i, 0))],
        out_shape=[jax.ShapeDtypeStruct((_EP, 8), f)],
    )(eap, W1bd, b1r, W2bd, b2r, W3bd, b3r)[0]

    ew_flat = ews.reshape(-1)
    ew = ew_flat[:E]
    ws = ew_flat[E]
    return z, log_q, beta, ew, ws


def kernel(x, edge_index, edge_attr, W1, b1, g1, be1, W2, b2, g2, be2, W3, b3,
           x0, ap, bp, mu, log_var, We1, eb1, We2, eb2, We3, eb3):
    z, log_q, beta, ew, ws = _dense_stage(
        x, W1, b1, g1, be1, W2, b2, g2, be2, W3, b3,
        x0, ap, bp, mu, log_var, We1, eb1, We2, eb2, We3, eb3, edge_attr)

    row = edge_index[0]
    col = edge_index[1]
    deg = jax.ops.segment_sum(ew, row, num_segments=N) + ws
    nrm = ew / (deg[row] + 1e-10)
    nrm_self = ws / (deg + 1e-10)

    hh = beta
    for _ in range(APPNP_K):
        msg = jax.ops.segment_sum(nrm[:, None] * hh[col], row, num_segments=N)
        msg = msg + nrm_self[:, None] * hh
        hh = (1.0 - APPNP_ALPHA) * msg + APPNP_ALPHA * beta

    alpha = BETA_PRIOR + jnp.maximum(hh, 0.0)
    probs = alpha / jnp.sum(alpha, axis=-1, keepdims=True)
    ew_out = jnp.concatenate([ew, jnp.full((N,), ws, _F32)])
    return (alpha, probs, z, beta, log_q, ew_out)


# speedup vs baseline: 19.2820x; 13.7337x over previous
import functools
import math

import jax
import jax.numpy as jnp
from jax import lax
from jax.experimental import pallas as pl
from jax.experimental.pallas import tpu as pltpu
from jax.experimental.pallas import tpu_sc as plsc

N = 10000
E = 320000
D_FEAT = 128
D_HID = 256
D_LAT = 16
N_CLASSES = 8
N_LAYERS = 10
EDGE_DIM = 16
EDGE_HID = 32
APPNP_K = 10
APPNP_ALPHA = 0.1
BETA_PRIOR = 0.001
LOG_SCALE = 0.5 * D_LAT * math.log(4 * math.pi)

_RN = 1000
_NBN = N // _RN
_EP = 41000
_RE = 1000
_NBE = _EP // _RE

_F32 = jnp.float32


def _softplus(v):
    return jnp.maximum(v, 0.0) + jnp.log1p(jnp.exp(-jnp.abs(v)))


def _enc1_body(x_ref, w_ref, b_ref, a_ref, s_ref):
    i = pl.program_id(0)

    @pl.when(i == 0)
    def _():
        s_ref[...] = jnp.zeros_like(s_ref)

    a = jnp.dot(x_ref[...], w_ref[...], preferred_element_type=_F32) + b_ref[...]
    a_ref[...] = a
    ssum = jnp.sum(a, axis=0, keepdims=True)
    ssq = jnp.sum(a * a, axis=0, keepdims=True)
    pad = jnp.zeros((6, a.shape[1]), _F32)
    s_ref[...] = s_ref[...] + jnp.concatenate([ssum, ssq, pad], axis=0)


def _enc2_body(a_ref, s_ref, g_ref, be_ref, w_ref, b_ref, a2_ref, s2_ref):
    i = pl.program_id(0)

    @pl.when(i == 0)
    def _():
        s2_ref[...] = jnp.zeros_like(s2_ref)

    mean = s_ref[0:1, :] / N
    var = s_ref[1:2, :] / N - mean * mean
    rstd = jax.lax.rsqrt(var + 1e-5)
    h = jnp.maximum((a_ref[...] - mean) * rstd * g_ref[...] + be_ref[...], 0.0)
    a2 = jnp.dot(h, w_ref[...], preferred_element_type=_F32) + b_ref[...]
    a2_ref[...] = a2
    ssum = jnp.sum(a2, axis=0, keepdims=True)
    ssq = jnp.sum(a2 * a2, axis=0, keepdims=True)
    pad = jnp.zeros((6, a2.shape[1]), _F32)
    s2_ref[...] = s2_ref[...] + jnp.concatenate([ssum, ssq, pad], axis=0)


def _flow_body(a_ref, s_ref, g_ref, be_ref, w_ref, b_ref, x0_ref, ap_ref,
               bp_ref, mu_ref, lv_ref, G_ref, H_ref, T_ref,
               z_ref, lq_ref, bt_ref, b09_ref, b01_ref):
    mean = s_ref[0:1, :] / N
    var = s_ref[1:2, :] / N - mean * mean
    rstd = jax.lax.rsqrt(var + 1e-5)
    h = jnp.maximum((a_ref[...] - mean) * rstd * g_ref[...] + be_ref[...], 0.0)
    z = jnp.dot(h, w_ref[...], preferred_element_type=_F32) + b_ref[...]
    z_ref[...] = z

    G = G_ref[...]
    H = H_ref[...]
    zz = jnp.dot(z, T_ref[...], preferred_element_type=_F32)
    ldsum = jnp.zeros((zz.shape[0], N_CLASSES), _F32)
    for l in range(N_LAYERS - 1, -1, -1):
        a8 = _softplus(ap_ref[l:l + 1, :])
        b8 = -a8 + _softplus(bp_ref[l:l + 1, :])
        diff = zz - x0_ref[l:l + 1, :]
        r2 = jnp.dot(diff * diff, G, preferred_element_type=_F32)
        r = jnp.maximum(jnp.sqrt(r2), 1e-8)
        hr = 1.0 / (a8 + r)
        bhr = b8 * hr
        zz = zz + jnp.dot(bhr, H, preferred_element_type=_F32) * diff
        ld = ((D_LAT - 1) * jnp.log1p(bhr)
              + jnp.log1p(bhr - b8 * hr * hr * r))
        ldsum = ldsum + ld
    lv = lv_ref[...]
    slv8 = jnp.dot(lv, G, preferred_element_type=_F32)
    dm = zz - mu_ref[...]
    q = jnp.dot(dm * dm * jnp.exp(-lv), G, preferred_element_type=_F32)
    logp = (-0.5 * D_LAT * math.log(2 * math.pi)) - 0.5 * slv8 - 0.5 * q
    log_q = logp + ldsum
    lq_ref[...] = log_q
    bt = jnp.exp(jnp.clip(log_q + LOG_SCALE, -30.0, 30.0))
    bt_ref[...] = bt
    b09_ref[...] = (1.0 - APPNP_ALPHA) * bt
    b01_ref[...] = APPNP_ALPHA * bt


def _edge_mlp_body(ea_ref, w1_ref, b1_ref, w2_ref, b2_ref, w3_ref, b3_ref, o_ref):
    h = jnp.maximum(jnp.dot(ea_ref[...], w1_ref[...], preferred_element_type=_F32)
                    + b1_ref[...], 0.0)
    h = jnp.maximum(jnp.dot(h, w2_ref[...], preferred_element_type=_F32)
                    + b2_ref[...], 0.0)
    lo = jnp.dot(h, w3_ref[...], preferred_element_type=_F32) + b3_ref[...]
    o_ref[...] = _softplus(lo)


def _dense_stage(x, W1, b1, g1, be1, W2, b2, g2, be2, W3, b3,
                 x0, ap, bp, mu, log_var, We1, eb1, We2, eb2, We3, eb3,
                 edge_attr):
    R = _RN
    f = _F32
    s_shape = jax.ShapeDtypeStruct((8, D_HID), f)

    a1, s1 = pl.pallas_call(
        _enc1_body,
        grid=(_NBN,),
        in_specs=[
            pl.BlockSpec((R, D_FEAT), lambda i: (i, 0)),
            pl.BlockSpec((D_FEAT, D_HID), lambda i: (0, 0)),
            pl.BlockSpec((1, D_HID), lambda i: (0, 0)),
        ],
        out_specs=[
            pl.BlockSpec((R, D_HID), lambda i: (i, 0)),
            pl.BlockSpec((8, D_HID), lambda i: (0, 0)),
        ],
        out_shape=[jax.ShapeDtypeStruct((N, D_HID), f), s_shape],
    )(x, W1, b1.reshape(1, -1))

    a2, s2 = pl.pallas_call(
        _enc2_body,
        grid=(_NBN,),
        in_specs=[
            pl.BlockSpec((R, D_HID), lambda i: (i, 0)),
            pl.BlockSpec((8, D_HID), lambda i: (0, 0)),
            pl.BlockSpec((1, D_HID), lambda i: (0, 0)),
            pl.BlockSpec((1, D_HID), lambda i: (0, 0)),
            pl.BlockSpec((D_HID, D_HID), lambda i: (0, 0)),
            pl.BlockSpec((1, D_HID), lambda i: (0, 0)),
        ],
        out_specs=[
            pl.BlockSpec((R, D_HID), lambda i: (i, 0)),
            pl.BlockSpec((8, D_HID), lambda i: (0, 0)),
        ],
        out_shape=[jax.ShapeDtypeStruct((N, D_HID), f), s_shape],
    )(a1, s1, g1.reshape(1, -1), be1.reshape(1, -1), W2, b2.reshape(1, -1))

    eye16 = jnp.eye(D_LAT, dtype=f)
    G = jnp.kron(jnp.eye(N_CLASSES, dtype=f), jnp.ones((D_LAT, 1), f))
    H = G.T
    T16 = jnp.kron(jnp.ones((1, N_CLASSES), f), eye16)
    x0p = jnp.transpose(x0, (1, 0, 2)).reshape(N_LAYERS, N_CLASSES * D_LAT)
    x0p = jnp.concatenate([x0p, jnp.zeros((16 - N_LAYERS, 128), f)], axis=0)
    app = jnp.concatenate([ap.T, jnp.zeros((16 - N_LAYERS, N_CLASSES), f)], 0)
    bpp = jnp.concatenate([bp.T, jnp.zeros((16 - N_LAYERS, N_CLASSES), f)], 0)
    mu128 = mu.reshape(1, -1)
    lv128 = log_var.reshape(1, -1)

    z, log_q, beta, b09, b01 = pl.pallas_call(
        _flow_body,
        grid=(_NBN,),
        in_specs=[
            pl.BlockSpec((R, D_HID), lambda i: (i, 0)),
            pl.BlockSpec((8, D_HID), lambda i: (0, 0)),
            pl.BlockSpec((1, D_HID), lambda i: (0, 0)),
            pl.BlockSpec((1, D_HID), lambda i: (0, 0)),
            pl.BlockSpec((D_HID, D_LAT), lambda i: (0, 0)),
            pl.BlockSpec((1, D_LAT), lambda i: (0, 0)),
            pl.BlockSpec((16, 128), lambda i: (0, 0)),
            pl.BlockSpec((16, N_CLASSES), lambda i: (0, 0)),
            pl.BlockSpec((16, N_CLASSES), lambda i: (0, 0)),
            pl.BlockSpec((1, 128), lambda i: (0, 0)),
            pl.BlockSpec((1, 128), lambda i: (0, 0)),
            pl.BlockSpec((128, N_CLASSES), lambda i: (0, 0)),
            pl.BlockSpec((N_CLASSES, 128), lambda i: (0, 0)),
            pl.BlockSpec((D_LAT, 128), lambda i: (0, 0)),
        ],
        out_specs=[
            pl.BlockSpec((R, D_LAT), lambda i: (i, 0)),
            pl.BlockSpec((R, N_CLASSES), lambda i: (i, 0)),
            pl.BlockSpec((R, N_CLASSES), lambda i: (i, 0)),
            pl.BlockSpec((R, N_CLASSES), lambda i: (i, 0)),
            pl.BlockSpec((R, N_CLASSES), lambda i: (i, 0)),
        ],
        out_shape=[
            jax.ShapeDtypeStruct((N, D_LAT), f),
            jax.ShapeDtypeStruct((N, N_CLASSES), f),
            jax.ShapeDtypeStruct((N, N_CLASSES), f),
            jax.ShapeDtypeStruct((N, N_CLASSES), f),
            jax.ShapeDtypeStruct((N, N_CLASSES), f),
        ],
    )(a2, s2, g2.reshape(1, -1), be2.reshape(1, -1), W3, b3.reshape(1, -1),
      x0p, app, bpp, mu128, lv128, G, H, T16)

    eap = edge_attr.reshape(E // 8, 8 * EDGE_DIM)
    eap = jnp.concatenate([eap, jnp.zeros((_EP - E // 8, 8 * EDGE_DIM), f)], 0)
    eye8 = jnp.eye(8, dtype=f)
    W1bd = jnp.kron(eye8, We1)
    W2bd = jnp.kron(eye8, We2)
    W3bd = jnp.kron(eye8, We3)
    b1r = jnp.tile(eb1, 8).reshape(1, -1)
    b2r = jnp.tile(eb2, 8).reshape(1, -1)
    b3r = jnp.tile(eb3, 8).reshape(1, -1)

    ews = pl.pallas_call(
        _edge_mlp_body,
        grid=(_NBE,),
        in_specs=[
            pl.BlockSpec((_RE, 128), lambda i: (i, 0)),
            pl.BlockSpec((128, 256), lambda i: (0, 0)),
            pl.BlockSpec((1, 256), lambda i: (0, 0)),
            pl.BlockSpec((256, 256), lambda i: (0, 0)),
            pl.BlockSpec((1, 256), lambda i: (0, 0)),
            pl.BlockSpec((256, 8), lambda i: (0, 0)),
            pl.BlockSpec((1, 8), lambda i: (0, 0)),
        ],
        out_specs=[pl.BlockSpec((_RE, 8), lambda i: (i, 0))],
        out_shape=[jax.ShapeDtypeStruct((_EP, 8), f)],
    )(eap, W1bd, b1r, W2bd, b2r, W3bd, b3r)[0]

    ew_flat = ews.reshape(-1)
    ew = ew_flat[:E]
    ws = ew_flat[E]
    return z, log_q, beta, b09, b01, ew, ws


_NP = 10240
_RPT = 640
_EB = 2688
_NEB = 8
_ECH = _EB * _NEB
_EPC = 16 * _ECH
_RPW = _NP // 16
_RDW = 2 * _NP // 16

_sc_mesh = plsc.VectorSubcoreMesh(core_axis_name="c", subcore_axis_name="s")


@functools.partial(
    pl.kernel,
    out_type=jax.ShapeDtypeStruct((2, 4, _NP), jnp.float32),
    mesh=_sc_mesh,
    compiler_params=pltpu.CompilerParams(needs_layout_passes=False),
    scratch_types=[
        pltpu.VMEM((_EB,), jnp.int32),
        pltpu.VMEM((_EB,), jnp.int32),
        pltpu.VMEM((_EB,), jnp.int32),
        pltpu.VMEM((_EB,), jnp.int32),
        pltpu.VMEM((_EB,), jnp.int32),
        pltpu.VMEM((_EB,), jnp.int32),
        pltpu.VMEM((_EB,), jnp.int32),
        pltpu.VMEM((_EB,), jnp.int32),
        pltpu.VMEM((_EB,), jnp.int32),
        pltpu.VMEM((_EB,), jnp.int32),
        pltpu.VMEM((_EB,), jnp.int32),
        pltpu.VMEM((_EB,), jnp.int32),
        pltpu.VMEM((_EB,), jnp.int32),
        pltpu.VMEM((_EB,), jnp.int32),
        pltpu.VMEM((_EB,), jnp.int32),
        pltpu.VMEM((_EB,), jnp.int32),
        pltpu.VMEM((_ECH,), jnp.float32),
        pltpu.VMEM((4 * _NP,), jnp.float32),
        pltpu.VMEM((_NP,), jnp.float32),
        pltpu.VMEM((_EB,), jnp.float32),
        pltpu.VMEM((_EB,), jnp.float32),
        pltpu.VMEM((4 * _RPW,), jnp.float32),
        pltpu.VMEM((_RPW,), jnp.float32),
        pltpu.VMEM((_RPW,), jnp.int32),
        pltpu.VMEM_SHARED((_NP,), jnp.float32),
        pltpu.VMEM_SHARED((_NP,), jnp.float32),
        pltpu.VMEM_SHARED((_NP,), jnp.float32),
        pltpu.VMEM_SHARED((_NP,), jnp.float32),
        pltpu.VMEM_SHARED((_NP,), jnp.float32),
        pltpu.VMEM_SHARED((_NP,), jnp.float32),
        pltpu.VMEM_SHARED((_NP,), jnp.float32),
        pltpu.VMEM_SHARED((_NP,), jnp.float32),
        pltpu.SemaphoreType.DMA,
        pltpu.SemaphoreType.DMA,
    ],
)
def _appnp_sc(col_h, row_h, ew_h, b09_h, b01_h, out_h,
              cb0, cb1, cb2, cb3, cb4, cb5, cb6, cb7,
              rb0, rb1, rb2, rb3, rb4, rb5, rb6, rb7,
              nrmb, hhl, dmab, msg_a, msg_b,
              betab, tmpb, myrowsb,
              hh0, hh1, hh2, hh3, acc0, acc1, acc2, acc3,
              sga, sgb):
    colb = (cb0, cb1, cb2, cb3, cb4, cb5, cb6, cb7)
    rowb = (rb0, rb1, rb2, rb3, rb4, rb5, rb6, rb7)
    c = lax.axis_index("c")
    s = lax.axis_index("s")
    r0 = s * _RPW
    iota = lax.iota(jnp.int32, 16)
    hhp = (hh0, hh1, hh2, hh3)
    accp = (acc0, acc1, acc2, acc3)

    for b in range(_NEB):
        pltpu.sync_copy(col_h.at[s, b], colb[b])
        pltpu.sync_copy(row_h.at[s, b], rowb[b])
    pltpu.sync_copy(ew_h.at[s], nrmb)
    for p in range(4):
        pltpu.sync_copy(b01_h.at[c, p, pl.ds(r0, _RPW)],
                        betab.at[pl.ds(p * _RPW, _RPW)])

    @plsc.parallel_loop(0, _RPW // 16, 1, unroll=8)
    def _(v):
        dmab[pl.ds(v * 16, 16)] = jnp.zeros((16,), jnp.float32)
        myrowsb[pl.ds(v * 16, 16)] = r0 + v * 16 + iota

    @plsc.parallel_loop(0, _RPW // 16, 1, unroll=8)
    def _(v):
        tmpb[pl.ds(v * 16, 16)] = jnp.zeros((16,), jnp.float32)

    pltpu.sync_copy(dmab.at[pl.ds(0, _RPW)], acc0.at[pl.ds(r0, _RPW)])
    plsc.subcore_barrier()

    for b in range(_NEB):
        pltpu.sync_copy(nrmb.at[pl.ds(b * _EB, _EB)],
                        acc0.at[rowb[b]], add=True)
    plsc.subcore_barrier()

    for b in range(_NEB):
        pltpu.async_copy(acc0.at[rowb[b]], dmab.at[pl.ds(0, _EB)], sga).wait()

        @plsc.parallel_loop(0, _EB // 16, 1, unroll=8)
        def _(v, _b=b):
            e0 = _b * _EB + v * 16
            ew16 = nrmb[pl.ds(e0, 16)]
            dg16 = dmab[pl.ds(v * 16, 16)]
            nrmb[pl.ds(e0, 16)] = ((1.0 - APPNP_ALPHA) * ew16
                                   / (dg16 + 1e-10))
    plsc.subcore_barrier()

    for p in range(4):
        pltpu.sync_copy(b09_h.at[c, p, pl.ds(r0, _RPW)],
                        accp[p].at[pl.ds(r0, _RPW)])

    def kbody(_, carry):
        for p in range(4):
            pltpu.sync_copy(accp[p].at[pl.ds(r0, _RPW)],
                            dmab.at[pl.ds(p * _RPW, _RPW)])
            pltpu.sync_copy(dmab.at[pl.ds(p * _RPW, _RPW)],
                            hhp[p].at[pl.ds(r0, _RPW)])
            pltpu.sync_copy(betab.at[pl.ds(p * _RPW, _RPW)],
                            hhp[p].at[myrowsb], add=True)
        plsc.subcore_barrier()

        for p in range(4):
            pltpu.sync_copy(hhp[p], dmab)

            @plsc.parallel_loop(0, _NP // 16, 1, unroll=8)
            def _(v, _p=p):
                hhl[pl.ds(_p * _NP + v * 16, 16)] = dmab[pl.ds(v * 16, 16)]

        for p in range(4):
            pltpu.sync_copy(tmpb, accp[p].at[pl.ds(r0, _RPW)])
        plsc.subcore_barrier()

        units = [(u >> 2, u & 3) for u in range(32)]
        bufs = (msg_a, msg_b)
        ssems = (sga, sgb)
        sds = {}
        for u, (b, f) in enumerate(units):
            if u >= 2:
                sds[u - 2].wait()
            buf = bufs[u & 1]

            @plsc.parallel_loop(0, _EB // 16, 1, unroll=4)
            def _(v, _b=b, _f=f, _buf=buf):
                col16 = colb[_b][pl.ds(v * 16, 16)]
                nr16 = nrmb[pl.ds(_b * _EB + v * 16, 16)]
                h = plsc.load_gather(hhl, [col16 + _f * _NP])
                _buf[pl.ds(v * 16, 16)] = h * nr16

            sds[u] = pltpu.async_copy(buf, accp[f].at[rowb[b]],
                                      ssems[u & 1], add=True)
        sds[30].wait()
        sds[31].wait()
        plsc.subcore_barrier()
        return carry

    lax.fori_loop(0, APPNP_K, kbody, 0)

    for p in range(4):
        pltpu.sync_copy(accp[p].at[pl.ds(r0, _RPW)],
                        out_h.at[c, p, pl.ds(r0, _RPW)])


def _final_body(acc_ref, beta_ref, alpha_ref, probs_ref):
    hh = acc_ref[...] + APPNP_ALPHA * beta_ref[...]
    alpha = BETA_PRIOR + jnp.maximum(hh, 0.0)
    alpha_ref[...] = alpha
    probs_ref[...] = alpha / jnp.sum(alpha, axis=1, keepdims=True)


def _planes(v):
    vp = jnp.concatenate([v, jnp.zeros((_NP - N, N_CLASSES), _F32)], 0)
    return vp.T.reshape(2, 4, _NP)


def kernel(x, edge_index, edge_attr, W1, b1, g1, be1, W2, b2, g2, be2, W3, b3,
           x0, ap, bp, mu, log_var, We1, eb1, We2, eb2, We3, eb3):
    f = _F32
    z, log_q, beta, b09, b01, ew, ws = _dense_stage(
        x, W1, b1, g1, be1, W2, b2, g2, be2, W3, b3,
        x0, ap, bp, mu, log_var, We1, eb1, We2, eb2, We3, eb3, edge_attr)

    row = edge_index[0]
    col = edge_index[1]
    sl = jnp.arange(N, dtype=row.dtype)
    padE = _EPC - E - N
    colf = jnp.concatenate([col, sl, jnp.zeros((padE,), row.dtype)])
    rowf = jnp.concatenate([row, sl, jnp.zeros((padE,), row.dtype)])
    ewf = jnp.concatenate([ew, jnp.full((N,), ws, f), jnp.zeros((padE,), f)])
    col_h = colf.reshape(16, _NEB, _EB)
    row_h = rowf.reshape(16, _NEB, _EB)
    ew_h = ewf.reshape(16, _ECH)
    b09_h = _planes(b09)
    b01_h = _planes(b01)

    acc = _appnp_sc(col_h, row_h, ew_h, b09_h, b01_h)
    acc8 = acc.reshape(8, _NP)[:, :N].T

    alpha, probs = pl.pallas_call(
        _final_body,
        grid=(_NBN,),
        in_specs=[
            pl.BlockSpec((_RN, N_CLASSES), lambda i: (i, 0)),
            pl.BlockSpec((_RN, N_CLASSES), lambda i: (i, 0)),
        ],
        out_specs=[
            pl.BlockSpec((_RN, N_CLASSES), lambda i: (i, 0)),
            pl.BlockSpec((_RN, N_CLASSES), lambda i: (i, 0)),
        ],
        out_shape=[
            jax.ShapeDtypeStruct((N, N_CLASSES), f),
            jax.ShapeDtypeStruct((N, N_CLASSES), f),
        ],
    )(acc8, beta)

    ew_out = jnp.concatenate([ew, jnp.full((N,), ws, f)])
    return (alpha, probs, z, beta, log_q, ew_out)
